# Initial kernel scaffold; baseline (speedup 1.0000x reference)
import jax
import jax.numpy as jnp
from jax.experimental import pallas as pl


def kernel(x, edge_index, edge_attr, u, batch, W0, b0, W1, b1):
    raise NotImplementedError("write your pallas kernel here")


# speedup vs baseline: 3.5995x; 3.5995x over previous
import functools

import jax
import jax.numpy as jnp
from jax import lax
from jax.experimental import pallas as pl
from jax.experimental.pallas import tpu as pltpu
from jax.experimental.pallas import tpu_sc as plsc

_N_NODES = 10000
_D = 128
_G = 64
_NC = 2
_NS = 16
_NW = _NC * _NS
_CHUNK = 320
_N_PAD = _CHUNK * _NW
_IDXW = 16
_NJ = _CHUNK // _IDXW
_HW = _G * 16 + 16


@functools.partial(
    pl.kernel,
    out_type=[
        jax.ShapeDtypeStruct((_NC, _G, _D), jnp.float32),
        jax.ShapeDtypeStruct((_NW * _HW,), jnp.float32),
    ],
    mesh=plsc.VectorSubcoreMesh(core_axis_name="c", subcore_axis_name="s"),
    compiler_params=pltpu.CompilerParams(needs_layout_passes=False),
    scratch_types=[
        pltpu.VMEM((_CHUNK, _D), jnp.float32),
        pltpu.VMEM((_CHUNK,), jnp.int32),
        pltpu.VMEM((_HW,), jnp.float32),
        pltpu.VMEM_SHARED((_G, _D), jnp.float32),
    ],
)
def _sc_segment(x_hbm, b_hbm, zx_hbm, zc_hbm, out_x, out_c,
                xbuf, idx, hist, acc_x):
    cid = lax.axis_index("c")
    sid = lax.axis_index("s")
    wid = sid * _NC + cid
    base = wid * _CHUNK

    @pl.when(sid == 0)
    def _zero():
        pltpu.sync_copy(zx_hbm, acc_x)

    pltpu.sync_copy(b_hbm.at[pl.ds(base, _CHUNK)], idx)
    pltpu.sync_copy(x_hbm.at[pl.ds(base, _CHUNK)], xbuf)
    pltpu.sync_copy(zc_hbm, hist)

    lanes = lax.iota(jnp.int32, 16)
    one16 = jnp.ones((16,), jnp.float32)
    valid = _N_NODES - base

    plsc.subcore_barrier()

    for j in range(_NJ):
        bv = idx[pl.ds(j * _IDXW, _IDXW)]
        pltpu.sync_copy(xbuf.at[pl.ds(j * _IDXW, _IDXW)],
                        acc_x.at[bv], add=True)
        ok = (j * _IDXW + lanes) < valid
        fidx = jnp.where(ok, bv * 16, _G * 16) + lanes
        plsc.addupdate_scatter(hist, [fidx], one16)

    pltpu.sync_copy(hist, out_c.at[pl.ds(wid * _HW, _HW)])

    plsc.subcore_barrier()

    @pl.when(sid == 0)
    def _dump():
        pltpu.sync_copy(acc_x, out_x.at[cid])


def _tc_body(px_ref, pc_ref, u_ref, w0_ref, b0_ref, w1_ref, b1_ref, o_ref):
    sums = px_ref[0] + px_ref[1]
    cnt = jnp.sum(jnp.sum(pc_ref[...], axis=0)[0:_G], axis=1, keepdims=True)
    mean = sums / jnp.maximum(cnt, 1.0)
    w0 = w0_ref[...]
    h = (jnp.dot(u_ref[...], w0[0:_D, :], preferred_element_type=jnp.float32,
                 precision=lax.Precision.HIGHEST)
         + jnp.dot(mean, w0[_D:2 * _D, :], preferred_element_type=jnp.float32,
                   precision=lax.Precision.HIGHEST)
         + b0_ref[...])
    h = jnp.maximum(h, 0.0)
    o_ref[...] = (jnp.dot(h, w1_ref[...], preferred_element_type=jnp.float32,
                          precision=lax.Precision.HIGHEST) + b1_ref[...])


_tc_mlp = pl.pallas_call(
    _tc_body,
    out_shape=jax.ShapeDtypeStruct((_G, _D), jnp.float32),
)


def kernel(x, edge_index, edge_attr, u, batch, W0, b0, W1, b1):
    del edge_index, edge_attr
    xp = jnp.pad(x, ((0, _N_PAD - _N_NODES), (0, 0)))
    bp = jnp.pad(batch, (0, _N_PAD - _N_NODES))
    zx_hbm = jnp.zeros((_G, _D), jnp.float32)
    zc_hbm = jnp.zeros((_HW,), jnp.float32)
    part_x, part_c = _sc_segment(xp, bp, zx_hbm, zc_hbm)
    part_c = part_c.reshape(_NW, _HW // 16, 16)
    return _tc_mlp(part_x, part_c, u, W0, b0.reshape(1, _D), W1,
                   b1.reshape(1, _D))


# speedup vs baseline: 3.9086x; 1.0859x over previous
import functools

import jax
import jax.numpy as jnp
from jax import lax
from jax.experimental import pallas as pl
from jax.experimental.pallas import tpu as pltpu
from jax.experimental.pallas import tpu_sc as plsc

_N_NODES = 10000
_D = 128
_G = 64
_NC = 2
_NS = 16
_NW = _NC * _NS
_CHUNK = 320
_N_PAD = _CHUNK * _NW
_IDXW = 16
_NJ = _CHUNK // _IDXW
_HW = _G * 16 + 16


@functools.partial(
    pl.kernel,
    out_type=[
        jax.ShapeDtypeStruct((_NC, _G, _D), jnp.float32),
        jax.ShapeDtypeStruct((_NW * _HW,), jnp.float32),
    ],
    mesh=plsc.VectorSubcoreMesh(core_axis_name="c", subcore_axis_name="s"),
    compiler_params=pltpu.CompilerParams(needs_layout_passes=False),
    scratch_types=[
        pltpu.VMEM((_CHUNK, _D), jnp.float32),
        pltpu.VMEM((_CHUNK,), jnp.int32),
        pltpu.VMEM((_HW,), jnp.float32),
        pltpu.VMEM_SHARED((_G, _D), jnp.float32),
        pltpu.SemaphoreType.DMA,
    ],
)
def _sc_segment(x_hbm, xtail_hbm, b_hbm, zx_hbm, zc_hbm, out_x, out_c,
                xbuf, idx, hist, acc_x, sem):
    cid = lax.axis_index("c")
    sid = lax.axis_index("s")
    wid = sid * _NC + cid
    base = wid * _CHUNK

    @pl.when(sid == 0)
    def _zero():
        pltpu.sync_copy(zx_hbm, acc_x)

    pltpu.sync_copy(b_hbm.at[pl.ds(base, _CHUNK)], idx)

    @pl.when(wid < _NW - 1)
    def _stage_main():
        pltpu.sync_copy(x_hbm.at[pl.ds(base, _CHUNK)], xbuf)

    @pl.when(wid == _NW - 1)
    def _stage_tail():
        pltpu.sync_copy(xtail_hbm, xbuf)

    pltpu.sync_copy(zc_hbm, hist)

    lanes = lax.iota(jnp.int32, 16)
    one16 = jnp.ones((16,), jnp.float32)
    valid = _N_NODES - base

    plsc.subcore_barrier()

    descs = []
    for j in range(_NJ):
        bv = idx[pl.ds(j * _IDXW, _IDXW)]
        descs.append(pltpu.async_copy(xbuf.at[pl.ds(j * _IDXW, _IDXW)],
                                      acc_x.at[bv], sem, add=True))
        ok = (j * _IDXW + lanes) < valid
        fidx = jnp.where(ok, bv * 16, _G * 16) + lanes
        plsc.addupdate_scatter(hist, [fidx], one16)

    pltpu.sync_copy(hist, out_c.at[pl.ds(wid * _HW, _HW)])
    for d in descs:
        d.wait()

    plsc.subcore_barrier()

    @pl.when(sid == 0)
    def _dump():
        pltpu.sync_copy(acc_x, out_x.at[cid])


def _tc_body(px_ref, pc_ref, u_ref, w0_ref, b0_ref, w1_ref, b1_ref, o_ref):
    sums = px_ref[0] + px_ref[1]
    cnt = jnp.sum(jnp.sum(pc_ref[...], axis=0)[0:_G], axis=1, keepdims=True)
    mean = sums / jnp.maximum(cnt, 1.0)
    w0 = w0_ref[...]
    h = (jnp.dot(u_ref[...], w0[0:_D, :], preferred_element_type=jnp.float32,
                 precision=lax.Precision.HIGHEST)
         + jnp.dot(mean, w0[_D:2 * _D, :], preferred_element_type=jnp.float32,
                   precision=lax.Precision.HIGHEST)
         + b0_ref[...])
    h = jnp.maximum(h, 0.0)
    o_ref[...] = (jnp.dot(h, w1_ref[...], preferred_element_type=jnp.float32,
                          precision=lax.Precision.HIGHEST) + b1_ref[...])


_tc_mlp = pl.pallas_call(
    _tc_body,
    out_shape=jax.ShapeDtypeStruct((_G, _D), jnp.float32),
)


def kernel(x, edge_index, edge_attr, u, batch, W0, b0, W1, b1):
    del edge_index, edge_attr
    tail_lo = (_NW - 1) * _CHUNK
    xtail = jnp.pad(x[tail_lo:], ((0, _N_PAD - _N_NODES), (0, 0)))
    bp = jnp.pad(batch, (0, _N_PAD - _N_NODES))
    zx_hbm = jnp.zeros((_G, _D), jnp.float32)
    zc_hbm = jnp.zeros((_HW,), jnp.float32)
    part_x, part_c = _sc_segment(x, xtail, bp, zx_hbm, zc_hbm)
    part_c = part_c.reshape(_NW, _HW // 16, 16)
    return _tc_mlp(part_x, part_c, u, W0, b0.reshape(1, _D), W1,
                   b1.reshape(1, _D))


# speedup vs baseline: 4.0967x; 1.0481x over previous
import functools

import jax
import jax.numpy as jnp
from jax import lax
from jax.experimental import pallas as pl
from jax.experimental.pallas import tpu as pltpu
from jax.experimental.pallas import tpu_sc as plsc

_N_NODES = 10000
_D = 128
_G = 64
_NC = 2
_NS = 16
_NW = _NC * _NS
_CHUNK = 320
_N_PAD = _CHUNK * _NW
_IDXW = 16
_NJ = _CHUNK // _IDXW
_HW = _G * 16 + 16
_GA = _G + 8


@functools.partial(
    pl.kernel,
    out_type=[
        jax.ShapeDtypeStruct((_NC, _GA, _D), jnp.float32),
        jax.ShapeDtypeStruct((_NW * _HW,), jnp.float32),
    ],
    mesh=plsc.VectorSubcoreMesh(core_axis_name="c", subcore_axis_name="s"),
    compiler_params=pltpu.CompilerParams(needs_layout_passes=False),
    scratch_types=[
        pltpu.VMEM((_CHUNK, _D), jnp.float32),
        pltpu.VMEM((_CHUNK,), jnp.int32),
        pltpu.VMEM((_HW,), jnp.float32),
        pltpu.VMEM_SHARED((_GA, _D), jnp.float32),
        pltpu.SemaphoreType.DMA,
    ],
)
def _sc_segment(x_hbm, b_hbm, zx_hbm, zc_hbm, out_x, out_c,
                xbuf, idx, hist, acc_x, sem):
    cid = lax.axis_index("c")
    sid = lax.axis_index("s")
    wid = sid * _NC + cid
    base = wid * _CHUNK
    valid = _N_NODES - base

    @pl.when(sid == 0)
    def _zero():
        pltpu.sync_copy(zx_hbm, acc_x)

    @pl.when(wid < _NW - 1)
    def _stage_main():
        pltpu.sync_copy(b_hbm.at[pl.ds(base, _CHUNK)], idx)
        pltpu.sync_copy(x_hbm.at[pl.ds(base, _CHUNK)], xbuf)

    @pl.when(wid == _NW - 1)
    def _stage_tail():
        pltpu.sync_copy(b_hbm.at[pl.ds(base, _N_NODES - (_NW - 1) * _CHUNK)],
                        idx.at[pl.ds(0, _N_NODES - (_NW - 1) * _CHUNK)])
        pltpu.sync_copy(x_hbm.at[pl.ds(base, _N_NODES - (_NW - 1) * _CHUNK)],
                        xbuf.at[pl.ds(0, _N_NODES - (_NW - 1) * _CHUNK)])

    pltpu.sync_copy(zc_hbm, hist)

    lanes = lax.iota(jnp.int32, 16)
    one16 = jnp.ones((16,), jnp.float32)

    plsc.subcore_barrier()

    descs = []
    for j in range(_NJ):
        bv = idx[pl.ds(j * _IDXW, _IDXW)]
        ok = (j * _IDXW + lanes) < valid
        bvx = jnp.where(ok, bv, _G)
        descs.append(pltpu.async_copy(xbuf.at[pl.ds(j * _IDXW, _IDXW)],
                                      acc_x.at[bvx], sem, add=True))
        fidx = jnp.where(ok, bv * 16, _G * 16) + lanes
        plsc.addupdate_scatter(hist, [fidx], one16)

    pltpu.sync_copy(hist, out_c.at[pl.ds(wid * _HW, _HW)])
    for d in descs:
        d.wait()

    plsc.subcore_barrier()

    @pl.when(sid == 0)
    def _dump():
        pltpu.sync_copy(acc_x, out_x.at[cid])


def _tc_body(px_ref, pc_ref, u_ref, w0_ref, b0_ref, w1_ref, b1_ref, o_ref):
    sums = px_ref[0, 0:_G] + px_ref[1, 0:_G]
    cnt = jnp.sum(jnp.sum(pc_ref[...], axis=0)[0:_G], axis=1, keepdims=True)
    mean = sums / jnp.maximum(cnt, 1.0)
    w0 = w0_ref[...]
    h = (jnp.dot(u_ref[...], w0[0:_D, :], preferred_element_type=jnp.float32,
                 precision=lax.Precision.HIGHEST)
         + jnp.dot(mean, w0[_D:2 * _D, :], preferred_element_type=jnp.float32,
                   precision=lax.Precision.HIGHEST)
         + b0_ref[...])
    h = jnp.maximum(h, 0.0)
    o_ref[...] = (jnp.dot(h, w1_ref[...], preferred_element_type=jnp.float32,
                          precision=lax.Precision.HIGHEST) + b1_ref[...])


_tc_mlp = pl.pallas_call(
    _tc_body,
    out_shape=jax.ShapeDtypeStruct((_G, _D), jnp.float32),
)


def kernel(x, edge_index, edge_attr, u, batch, W0, b0, W1, b1):
    del edge_index, edge_attr
    zx_hbm = jnp.zeros((_GA, _D), jnp.float32)
    zc_hbm = jnp.zeros((_HW,), jnp.float32)
    part_x, part_c = _sc_segment(x, batch, zx_hbm, zc_hbm)
    part_c = part_c.reshape(_NW, _HW // 16, 16)
    return _tc_mlp(part_x, part_c, u, W0, b0.reshape(1, _D), W1,
                   b1.reshape(1, _D))


# speedup vs baseline: 4.3235x; 1.0554x over previous
import functools

import jax
import jax.numpy as jnp
import numpy as np
from jax import lax
from jax.experimental import pallas as pl
from jax.experimental.pallas import tpu as pltpu
from jax.experimental.pallas import tpu_sc as plsc

_N_NODES = 10000
_D = 128
_G = 64
_NC = 2
_NS = 16
_NW = _NC * _NS
_CHUNK = 320
_N_PAD = _CHUNK * _NW
_IDXW = 16
_NJ = _CHUNK // _IDXW
_HW = _G * 16 + 16
_GA = _G + 8
_ZX = np.zeros((_GA, _D), np.float32)
_ZC = np.zeros((_G + 1, 16), np.float32)


@functools.partial(
    pl.kernel,
    out_type=[
        jax.ShapeDtypeStruct((_NC, _GA, _D), jnp.float32),
        jax.ShapeDtypeStruct((_NW, _G + 1, 16), jnp.float32),
    ],
    mesh=plsc.VectorSubcoreMesh(core_axis_name="c", subcore_axis_name="s"),
    compiler_params=pltpu.CompilerParams(needs_layout_passes=False),
    scratch_types=[
        pltpu.VMEM((_CHUNK, _D), jnp.float32),
        pltpu.VMEM((_CHUNK,), jnp.int32),
        pltpu.VMEM((_G + 1, 16), jnp.float32),
        pltpu.VMEM_SHARED((_GA, _D), jnp.float32),
        pltpu.SemaphoreType.DMA,
    ],
)
def _sc_segment(x_hbm, b_hbm, zx_hbm, zc_hbm, out_x, out_c,
                xbuf, idx, hist, acc_x, sem):
    cid = lax.axis_index("c")
    sid = lax.axis_index("s")
    wid = sid * _NC + cid
    base = wid * _CHUNK
    valid = _N_NODES - base

    @pl.when(sid == 0)
    def _zero():
        pltpu.sync_copy(zx_hbm, acc_x)

    @pl.when(wid < _NW - 1)
    def _stage_main():
        pltpu.sync_copy(b_hbm.at[pl.ds(base, _CHUNK)], idx)
        pltpu.sync_copy(x_hbm.at[pl.ds(base, _CHUNK)], xbuf)

    @pl.when(wid == _NW - 1)
    def _stage_tail():
        pltpu.sync_copy(b_hbm.at[pl.ds(base, _N_NODES - (_NW - 1) * _CHUNK)],
                        idx.at[pl.ds(0, _N_NODES - (_NW - 1) * _CHUNK)])
        pltpu.sync_copy(x_hbm.at[pl.ds(base, _N_NODES - (_NW - 1) * _CHUNK)],
                        xbuf.at[pl.ds(0, _N_NODES - (_NW - 1) * _CHUNK)])

    pltpu.sync_copy(zc_hbm, hist)

    lanes = lax.iota(jnp.int32, 16)
    one16 = jnp.ones((16,), jnp.float32)

    plsc.subcore_barrier()

    descs = []
    for j in range(_NJ):
        bv = idx[pl.ds(j * _IDXW, _IDXW)]
        ok = (j * _IDXW + lanes) < valid
        bvx = jnp.where(ok, bv, _G)
        descs.append(pltpu.async_copy(xbuf.at[pl.ds(j * _IDXW, _IDXW)],
                                      acc_x.at[bvx], sem, add=True))
        plsc.addupdate_scatter(hist, [bvx, lanes], one16)

    pltpu.sync_copy(hist, out_c.at[wid])
    for d in descs:
        d.wait()

    plsc.subcore_barrier()

    @pl.when(sid == 0)
    def _dump():
        pltpu.sync_copy(acc_x, out_x.at[cid])


def _tc_body(px_ref, pc_ref, u_ref, w0_ref, b0_ref, w1_ref, b1_ref, o_ref):
    sums = px_ref[0, 0:_G] + px_ref[1, 0:_G]
    cnt = jnp.sum(jnp.sum(pc_ref[:, 0:_G, :], axis=0), axis=1, keepdims=True)
    mean = sums / jnp.maximum(cnt, 1.0)
    w0 = w0_ref[...]
    h = (jnp.dot(u_ref[...], w0[0:_D, :], preferred_element_type=jnp.float32,
                 precision=lax.Precision.HIGHEST)
         + jnp.dot(mean, w0[_D:2 * _D, :], preferred_element_type=jnp.float32,
                   precision=lax.Precision.HIGHEST)
         + b0_ref[...])
    h = jnp.maximum(h, 0.0)
    o_ref[...] = (jnp.dot(h, w1_ref[...], preferred_element_type=jnp.float32,
                          precision=lax.Precision.HIGHEST) + b1_ref[...])


_tc_mlp = pl.pallas_call(
    _tc_body,
    out_shape=jax.ShapeDtypeStruct((_G, _D), jnp.float32),
)


def kernel(x, edge_index, edge_attr, u, batch, W0, b0, W1, b1):
    del edge_index, edge_attr
    part_x, part_c = _sc_segment(x, batch, _ZX, _ZC)
    return _tc_mlp(part_x, part_c, u, W0, b0.reshape(1, _D), W1,
                   b1.reshape(1, _D))
